# baseline (speedup 1.0000x reference)
import functools

import jax
import jax.numpy as jnp
from jax import lax
from jax.experimental import pallas as pl
from jax.experimental.pallas import tpu as pltpu

WIN_H = 72
WIN_W = 128
NCHUNK = 16


def _roi_kernel(order, ibx, rya, ryb, lyv, feat_hbm, mx7, mx14, out7, out14,
                band, sems, state):
    i = pl.program_id(0)
    ro = order[i]
    b = ibx[0, ro]
    y0 = ibx[1, ro]
    x0 = pl.multiple_of(ibx[2, ro], 16)

    @pl.when(i == 0)
    def _():
        state[0] = -1
        state[1] = 0

    reset = b != state[0]
    start_chunk = jnp.where(reset, y0 // 8, state[1])
    end_chunk = (y0 + WIN_H + 7) // 8

    def load_chunk(c, _):
        slot = lax.rem(c, NCHUNK)
        cp = pltpu.make_async_copy(
            feat_hbm.at[b, pl.ds(c * 8, 8), :, :],
            band.at[slot], sems.at[slot])
        cp.start()
        cp.wait()
        return 0

    lax.fori_loop(start_chunk, end_chunk, load_chunk, 0)
    state[0] = b
    state[1] = jnp.maximum(end_chunk, start_chunk)

    def row_slice(a):
        slot = lax.rem(a // 8, NCHUNK)
        return band[slot, lax.rem(a, 8), pl.ds(x0, WIN_W), :]

    def pooled_row(k1):
        acc = None
        for k in (k1, k1 + 1):
            la = lyv[k, ro].astype(jnp.bfloat16)
            rowa = row_slice(rya[k, ro])
            rowb = row_slice(ryb[k, ro])
            contrib = rowa + la * (rowb - rowa)
            acc = contrib if acc is None else acc + contrib
        return jnp.bfloat16(0.5) * acc

    m7 = mx7[0]
    m14 = mx14[0]
    for q in range(7):
        t = pooled_row(2 * q)
        out7[0, q] = jnp.dot(m7, t, preferred_element_type=jnp.float32)[:7]
    for q in range(14):
        t = pooled_row(14 + 2 * q)
        out14[0, q] = jnp.dot(
            m14, t, preferred_element_type=jnp.float32)[:14].astype(jnp.bfloat16)


def _build_mx(rxa, rxb, lx, p):
    iota = jnp.arange(WIN_W, dtype=jnp.int32)
    oa = ((iota[None, None, :] == rxa[:, :, None]) * (1.0 - lx)[:, :, None]
          + (iota[None, None, :] == rxb[:, :, None]) * lx[:, :, None])
    m = 0.5 * (oa[:, 0::2] + oa[:, 1::2])
    p_pad = 8 if p == 7 else 16
    return jnp.pad(m, ((0, 0), (0, p_pad - p), (0, 0))).astype(jnp.bfloat16)


def _roi_align_pallas(feat, boxes, batch_idx, *, interpret=False):
    N = boxes.shape[0]
    B, C, H, W = feat.shape
    feat_t = feat.transpose(0, 2, 3, 1).astype(jnp.bfloat16)

    x1, y1, x2, y2 = boxes[:, 0], boxes[:, 1], boxes[:, 2], boxes[:, 3]
    y0w = jnp.clip(jnp.floor(y1).astype(jnp.int32), 0, H - WIN_H)
    x0w = jnp.clip((jnp.floor(x1).astype(jnp.int32) // 16) * 16, 0, W - WIN_W)

    def samples(v1, v2, P, vmax):
        steps = (jnp.arange(P, dtype=jnp.float32) + 0.5) / P
        return jnp.clip(v1[:, None] + steps[None, :] * (v2 - v1)[:, None],
                        0.0, vmax)

    ys = jnp.concatenate([samples(y1, y2, 14, H - 1.0),
                          samples(y1, y2, 28, H - 1.0)], axis=1)
    ry0 = jnp.floor(ys).astype(jnp.int32)
    rya = ry0
    ryb = jnp.minimum(ry0 + 1, H - 1)
    lyv = ys - jnp.floor(ys)

    xs7 = samples(x1, x2, 14, W - 1.0)
    xs14 = samples(x1, x2, 28, W - 1.0)

    def xparts(xs):
        rx0 = jnp.floor(xs).astype(jnp.int32)
        return (rx0 - x0w[:, None], jnp.minimum(rx0 + 1, W - 1) - x0w[:, None],
                xs - jnp.floor(xs))

    mx7 = _build_mx(*xparts(xs7), 7)
    mx14 = _build_mx(*xparts(xs14), 14)

    bix = batch_idx.astype(jnp.int32)
    order = jnp.argsort(bix * 256 + y0w).astype(jnp.int32)
    ibx = jnp.stack([bix, y0w, x0w])
    ryaT = rya.T.astype(jnp.int32)
    rybT = ryb.T.astype(jnp.int32)
    lyT = lyv.T.astype(jnp.float32)

    out7, out14 = pl.pallas_call(
        _roi_kernel,
        grid_spec=pltpu.PrefetchScalarGridSpec(
            num_scalar_prefetch=5,
            grid=(N,),
            in_specs=[
                pl.BlockSpec(memory_space=pl.ANY),
                pl.BlockSpec((1, 8, WIN_W),
                             lambda i, order, *_: (order[i], 0, 0)),
                pl.BlockSpec((1, 16, WIN_W),
                             lambda i, order, *_: (order[i], 0, 0)),
            ],
            out_specs=[
                pl.BlockSpec((1, 7, 7, C),
                             lambda i, order, *_: (order[i], 0, 0, 0)),
                pl.BlockSpec((1, 14, 14, C),
                             lambda i, order, *_: (order[i], 0, 0, 0)),
            ],
            scratch_shapes=[
                pltpu.VMEM((NCHUNK, 8, W, C), jnp.bfloat16),
                pltpu.SemaphoreType.DMA((NCHUNK,)),
                pltpu.SMEM((2,), jnp.int32),
            ],
        ),
        out_shape=[
            jax.ShapeDtypeStruct((N, 7, 7, C), jnp.float32),
            jax.ShapeDtypeStruct((N, 14, 14, C), jnp.bfloat16),
        ],
        compiler_params=pltpu.CompilerParams(
            dimension_semantics=("arbitrary",),
            vmem_limit_bytes=50 * 1024 * 1024,
        ),
        name="roi_align",
        interpret=interpret,
    )(order, ibx, ryaT, rybT, lyT, feat_t, mx7, mx14)
    return out7, out14


PADROWS = 272
PBASE = 24


def _mask_kernel(xin, wconv, wdc, wpr, om, xp):
    def refill(src_rows):
        xp[...] = jnp.zeros_like(xp)
        for i in range(14):
            xp[PBASE + 16 * i: PBASE + 16 * i + 14] = src_rows(i)

    refill(lambda i: xin[0, 14 * i: 14 * i + 14, :])

    hb = None
    for l in range(4):
        xs9 = jnp.concatenate(
            [xp[PBASE + 16 * (k // 3 - 1) + (k % 3 - 1):
                PBASE + 16 * (k // 3 - 1) + (k % 3 - 1) + 224]
             for k in range(9)], axis=1)
        acc = jnp.dot(xs9, wconv[l], preferred_element_type=jnp.float32)
        hb = jax.nn.relu(acc).astype(jnp.bfloat16)
        if l < 3:
            refill(lambda i: hb[16 * i: 16 * i + 14])

    for ab in range(4):
        z = jax.nn.relu(
            jnp.dot(hb, wdc[ab], preferred_element_type=jnp.float32)
        ).astype(jnp.bfloat16)
        om[0, ab] = jnp.dot(z, wpr[...], preferred_element_type=jnp.float32)


def _mask_head_pallas(out14, m1, m2, m3, m4, mdc, mpr, *, interpret=False):
    N = out14.shape[0]
    xin = out14.reshape(N, 196, 256)
    wconv = jnp.stack([m.transpose(2, 3, 1, 0).reshape(2304, 256)
                       for m in (m1, m2, m3, m4)]).astype(jnp.bfloat16)
    wdc = mdc[:, :, ::-1, ::-1].transpose(2, 3, 0, 1).reshape(
        4, 256, 256).astype(jnp.bfloat16)
    wpr = jnp.pad(mpr.reshape(81, 256).T, ((0, 0), (0, 7))).astype(jnp.bfloat16)

    om = pl.pallas_call(
        _mask_kernel,
        grid=(N,),
        in_specs=[
            pl.BlockSpec((1, 196, 256), lambda i: (i, 0, 0)),
            pl.BlockSpec((4, 2304, 256), lambda i: (0, 0, 0)),
            pl.BlockSpec((4, 256, 256), lambda i: (0, 0, 0)),
            pl.BlockSpec((256, 88), lambda i: (0, 0)),
        ],
        out_specs=pl.BlockSpec((1, 4, 224, 88), lambda i: (i, 0, 0, 0)),
        scratch_shapes=[pltpu.VMEM((PADROWS, 256), jnp.bfloat16)],
        out_shape=jax.ShapeDtypeStruct((N, 4, 224, 88), jnp.float32),
        compiler_params=pltpu.CompilerParams(
            dimension_semantics=("arbitrary",),
            vmem_limit_bytes=50 * 1024 * 1024,
        ),
        name="mask_head",
        interpret=interpret,
    )(xin, wconv, wdc, wpr)

    om = om.reshape(N, 2, 2, 14, 16, 88)[:, :, :, :, :14, :81]
    return om.transpose(0, 5, 3, 1, 4, 2).reshape(N, 81, 28, 28)


def kernel(feat, boxes, bw1, bw2, bwo, cw1, cw2, cwo, m1, m2, m3, m4, mdc, mpr, batch_idx):
    out7, out14 = _roi_align_pallas(feat, boxes, batch_idx)
    flat = out7.transpose(0, 3, 1, 2).reshape(out7.shape[0], -1)
    box_result = jax.nn.relu(jax.nn.relu(flat @ bw1) @ bw2) @ bwo
    cls_result = jax.nn.relu(jax.nn.relu(flat @ cw1) @ cw2) @ cwo
    mask_result = _mask_head_pallas(out14, m1, m2, m3, m4, mdc, mpr)
    return (box_result, cls_result, mask_result)


# speedup vs baseline: 1.7649x; 1.7649x over previous
import functools

import jax
import jax.numpy as jnp
from jax import lax
from jax.experimental import pallas as pl
from jax.experimental.pallas import tpu as pltpu

WIN_H = 72
WIN_W = 128
NCHUNK = 16


def _roi_kernel(order, ibx, rya, ryb, lyv, feat_hbm, mx7, mx14, out7, out14,
                band, sems, state):
    i = pl.program_id(0)
    ro = order[i]
    b = ibx[0, ro]
    y0 = ibx[1, ro]
    x0 = pl.multiple_of(ibx[2, ro], 16)

    @pl.when(i == 0)
    def _():
        state[0] = -1
        state[1] = 0

    reset = b != state[0]
    start_chunk = jnp.where(reset, y0 // 8, state[1])
    end_chunk = (y0 + WIN_H + 7) // 8

    def load_chunk(c, _):
        slot = lax.rem(c, NCHUNK)
        cp = pltpu.make_async_copy(
            feat_hbm.at[b, pl.ds(c * 8, 8), :, :],
            band.at[slot], sems.at[slot])
        cp.start()
        cp.wait()
        return 0

    lax.fori_loop(start_chunk, end_chunk, load_chunk, 0)
    state[0] = b
    state[1] = jnp.maximum(end_chunk, start_chunk)

    def row_slice(a):
        slot = lax.rem(a // 8, NCHUNK)
        return band[slot, lax.rem(a, 8), pl.ds(x0, WIN_W), :]

    def pooled_row(k1):
        acc = None
        for k in (k1, k1 + 1):
            la = lyv[k, ro].astype(jnp.bfloat16)
            rowa = row_slice(rya[k, ro])
            rowb = row_slice(ryb[k, ro])
            contrib = rowa + la * (rowb - rowa)
            acc = contrib if acc is None else acc + contrib
        return jnp.bfloat16(0.5) * acc

    m7 = mx7[0]
    m14 = mx14[0]
    for q in range(7):
        t = pooled_row(2 * q)
        out7[0, q] = jnp.dot(m7, t, preferred_element_type=jnp.float32)[:7]
    for q in range(14):
        t = pooled_row(14 + 2 * q)
        out14[0, q] = jnp.dot(
            m14, t, preferred_element_type=jnp.float32)[:14].astype(jnp.bfloat16)


def _build_mx(rxa, rxb, lx, p):
    iota = jnp.arange(WIN_W, dtype=jnp.int32)
    oa = ((iota[None, None, :] == rxa[:, :, None]) * (1.0 - lx)[:, :, None]
          + (iota[None, None, :] == rxb[:, :, None]) * lx[:, :, None])
    m = 0.5 * (oa[:, 0::2] + oa[:, 1::2])
    p_pad = 8 if p == 7 else 16
    return jnp.pad(m, ((0, 0), (0, p_pad - p), (0, 0))).astype(jnp.bfloat16)


def _roi_align_pallas(feat, boxes, batch_idx, *, interpret=False):
    N = boxes.shape[0]
    B, C, H, W = feat.shape
    feat_t = feat.transpose(0, 2, 3, 1).astype(jnp.bfloat16)

    x1, y1, x2, y2 = boxes[:, 0], boxes[:, 1], boxes[:, 2], boxes[:, 3]
    y0w = jnp.clip(jnp.floor(y1).astype(jnp.int32), 0, H - WIN_H)
    x0w = jnp.clip((jnp.floor(x1).astype(jnp.int32) // 16) * 16, 0, W - WIN_W)

    def samples(v1, v2, P, vmax):
        steps = (jnp.arange(P, dtype=jnp.float32) + 0.5) / P
        return jnp.clip(v1[:, None] + steps[None, :] * (v2 - v1)[:, None],
                        0.0, vmax)

    ys = jnp.concatenate([samples(y1, y2, 14, H - 1.0),
                          samples(y1, y2, 28, H - 1.0)], axis=1)
    ry0 = jnp.floor(ys).astype(jnp.int32)
    rya = ry0
    ryb = jnp.minimum(ry0 + 1, H - 1)
    lyv = ys - jnp.floor(ys)

    xs7 = samples(x1, x2, 14, W - 1.0)
    xs14 = samples(x1, x2, 28, W - 1.0)

    def xparts(xs):
        rx0 = jnp.floor(xs).astype(jnp.int32)
        return (rx0 - x0w[:, None], jnp.minimum(rx0 + 1, W - 1) - x0w[:, None],
                xs - jnp.floor(xs))

    mx7 = _build_mx(*xparts(xs7), 7)
    mx14 = _build_mx(*xparts(xs14), 14)

    bix = batch_idx.astype(jnp.int32)
    order = jnp.argsort(bix * 256 + y0w).astype(jnp.int32)
    ibx = jnp.stack([bix, y0w, x0w])
    ryaT = rya.T.astype(jnp.int32)
    rybT = ryb.T.astype(jnp.int32)
    lyT = lyv.T.astype(jnp.float32)

    out7, out14 = pl.pallas_call(
        _roi_kernel,
        grid_spec=pltpu.PrefetchScalarGridSpec(
            num_scalar_prefetch=5,
            grid=(N,),
            in_specs=[
                pl.BlockSpec(memory_space=pl.ANY),
                pl.BlockSpec((1, 8, WIN_W),
                             lambda i, order, *_: (order[i], 0, 0)),
                pl.BlockSpec((1, 16, WIN_W),
                             lambda i, order, *_: (order[i], 0, 0)),
            ],
            out_specs=[
                pl.BlockSpec((1, 7, 7, C),
                             lambda i, order, *_: (order[i], 0, 0, 0)),
                pl.BlockSpec((1, 14, 14, C),
                             lambda i, order, *_: (order[i], 0, 0, 0)),
            ],
            scratch_shapes=[
                pltpu.VMEM((NCHUNK, 8, W, C), jnp.bfloat16),
                pltpu.SemaphoreType.DMA((NCHUNK,)),
                pltpu.SMEM((2,), jnp.int32),
            ],
        ),
        out_shape=[
            jax.ShapeDtypeStruct((N, 7, 7, C), jnp.float32),
            jax.ShapeDtypeStruct((N, 14, 14, C), jnp.bfloat16),
        ],
        compiler_params=pltpu.CompilerParams(
            dimension_semantics=("arbitrary",),
            vmem_limit_bytes=50 * 1024 * 1024,
        ),
        name="roi_align",
        interpret=interpret,
    )(order, ibx, ryaT, rybT, lyT, feat_t, mx7, mx14)
    return out7, out14


def _conv(x, w, pad='SAME'):
    return lax.conv_general_dilated(x, w, (1, 1), pad,
                                    dimension_numbers=('NCHW', 'OIHW', 'NCHW'))


def kernel(feat, boxes, bw1, bw2, bwo, cw1, cw2, cwo, m1, m2, m3, m4, mdc, mpr, batch_idx):
    out7, out14 = _roi_align_pallas(feat, boxes, batch_idx)
    flat = out7.transpose(0, 3, 1, 2).reshape(out7.shape[0], -1)
    box_result = jax.nn.relu(jax.nn.relu(flat @ bw1) @ bw2) @ bwo
    cls_result = jax.nn.relu(jax.nn.relu(flat @ cw1) @ cw2) @ cwo
    x = out14.astype(jnp.float32).transpose(0, 3, 1, 2)
    for w in (m1, m2, m3, m4):
        x = jax.nn.relu(_conv(x, w))
    x = jax.nn.relu(lax.conv_transpose(x, mdc, (2, 2), 'VALID',
                                       dimension_numbers=('NCHW', 'IOHW', 'NCHW')))
    mask_result = _conv(x, mpr)
    return (box_result, cls_result, mask_result)


# speedup vs baseline: 1.9700x; 1.1162x over previous
import functools

import jax
import jax.numpy as jnp
from jax import lax
from jax.experimental import pallas as pl
from jax.experimental.pallas import tpu as pltpu

WIN_H = 72
WIN_W = 128
NCHUNK = 16


def _roi_kernel(order, ibx, rya, ryb, lyv, feat_hbm, mx7, mx14, out7, out14,
                band, sems, state):
    i = pl.program_id(0)
    ro = order[i]
    b = ibx[0, ro]
    y0 = ibx[1, ro]
    x0 = pl.multiple_of(ibx[2, ro], 16)

    @pl.when(i == 0)
    def _():
        state[0] = -1
        state[1] = 0

    reset = b != state[0]
    start_chunk = jnp.where(reset, y0 // 8, state[1])
    end_chunk = (y0 + WIN_H + 7) // 8

    def load_chunk(c, _):
        slot = lax.rem(c, NCHUNK)
        cp = pltpu.make_async_copy(
            feat_hbm.at[b, pl.ds(c * 8, 8), :, :],
            band.at[pl.ds(slot * 8, 8)], sems.at[slot])
        cp.start()
        cp.wait()
        return 0

    lax.fori_loop(start_chunk, end_chunk, load_chunk, 0)
    state[0] = b
    state[1] = jnp.maximum(end_chunk, start_chunk)

    def row_slice(a):
        return band[lax.rem(a, NCHUNK * 8), pl.ds(x0, WIN_W), :]

    def pooled_row(k1):
        acc = None
        for k in (k1, k1 + 1):
            la = lyv[k, ro].astype(jnp.bfloat16)
            rowa = row_slice(rya[k, ro])
            rowb = row_slice(ryb[k, ro])
            contrib = rowa + la * (rowb - rowa)
            acc = contrib if acc is None else acc + contrib
        return jnp.bfloat16(0.5) * acc

    m7 = mx7[0]
    m14 = mx14[0]
    for q in range(7):
        t = pooled_row(2 * q)
        out7[0, q] = jnp.dot(m7, t, preferred_element_type=jnp.float32)[:7]
    for q in range(14):
        t = pooled_row(14 + 2 * q)
        out14[0, q] = jnp.dot(
            m14, t, preferred_element_type=jnp.float32)[:14].astype(jnp.bfloat16)


def _build_mx(rxa, rxb, lx, p):
    iota = jnp.arange(WIN_W, dtype=jnp.int32)
    oa = ((iota[None, None, :] == rxa[:, :, None]) * (1.0 - lx)[:, :, None]
          + (iota[None, None, :] == rxb[:, :, None]) * lx[:, :, None])
    m = 0.5 * (oa[:, 0::2] + oa[:, 1::2])
    p_pad = 8 if p == 7 else 16
    return jnp.pad(m, ((0, 0), (0, p_pad - p), (0, 0))).astype(jnp.bfloat16)


def _roi_align_pallas(feat, boxes, batch_idx, *, interpret=False):
    N = boxes.shape[0]
    B, C, H, W = feat.shape
    feat_t = feat.transpose(0, 2, 3, 1).astype(jnp.bfloat16)

    x1, y1, x2, y2 = boxes[:, 0], boxes[:, 1], boxes[:, 2], boxes[:, 3]
    y0w = jnp.clip(jnp.floor(y1).astype(jnp.int32), 0, H - WIN_H)
    x0w = jnp.clip((jnp.floor(x1).astype(jnp.int32) // 16) * 16, 0, W - WIN_W)

    def samples(v1, v2, P, vmax):
        steps = (jnp.arange(P, dtype=jnp.float32) + 0.5) / P
        return jnp.clip(v1[:, None] + steps[None, :] * (v2 - v1)[:, None],
                        0.0, vmax)

    ys = jnp.concatenate([samples(y1, y2, 14, H - 1.0),
                          samples(y1, y2, 28, H - 1.0)], axis=1)
    ry0 = jnp.floor(ys).astype(jnp.int32)
    rya = ry0
    ryb = jnp.minimum(ry0 + 1, H - 1)
    lyv = ys - jnp.floor(ys)

    xs7 = samples(x1, x2, 14, W - 1.0)
    xs14 = samples(x1, x2, 28, W - 1.0)

    def xparts(xs):
        rx0 = jnp.floor(xs).astype(jnp.int32)
        return (rx0 - x0w[:, None], jnp.minimum(rx0 + 1, W - 1) - x0w[:, None],
                xs - jnp.floor(xs))

    mx7 = _build_mx(*xparts(xs7), 7)
    mx14 = _build_mx(*xparts(xs14), 14)

    bix = batch_idx.astype(jnp.int32)
    order = jnp.argsort(bix * 256 + y0w).astype(jnp.int32)
    ibx = jnp.stack([bix, y0w, x0w])
    ryaT = rya.T.astype(jnp.int32)
    rybT = ryb.T.astype(jnp.int32)
    lyT = lyv.T.astype(jnp.float32)

    out7, out14 = pl.pallas_call(
        _roi_kernel,
        grid_spec=pltpu.PrefetchScalarGridSpec(
            num_scalar_prefetch=5,
            grid=(N,),
            in_specs=[
                pl.BlockSpec(memory_space=pl.ANY),
                pl.BlockSpec((1, 8, WIN_W),
                             lambda i, order, *_: (order[i], 0, 0)),
                pl.BlockSpec((1, 16, WIN_W),
                             lambda i, order, *_: (order[i], 0, 0)),
            ],
            out_specs=[
                pl.BlockSpec((1, 7, 7, C),
                             lambda i, order, *_: (order[i], 0, 0, 0)),
                pl.BlockSpec((1, 14, 14, C),
                             lambda i, order, *_: (order[i], 0, 0, 0)),
            ],
            scratch_shapes=[
                pltpu.VMEM((NCHUNK * 8, W, C), jnp.bfloat16),
                pltpu.SemaphoreType.DMA((NCHUNK,)),
                pltpu.SMEM((2,), jnp.int32),
            ],
        ),
        out_shape=[
            jax.ShapeDtypeStruct((N, 7, 7, C), jnp.float32),
            jax.ShapeDtypeStruct((N, 14, 14, C), jnp.bfloat16),
        ],
        compiler_params=pltpu.CompilerParams(
            dimension_semantics=("arbitrary",),
            vmem_limit_bytes=50 * 1024 * 1024,
        ),
        name="roi_align",
        interpret=interpret,
    )(order, ibx, ryaT, rybT, lyT, feat_t, mx7, mx14)
    return out7, out14


def _conv(x, w, pad='SAME'):
    return lax.conv_general_dilated(x, w, (1, 1), pad,
                                    dimension_numbers=('NCHW', 'OIHW', 'NCHW'))


def kernel(feat, boxes, bw1, bw2, bwo, cw1, cw2, cwo, m1, m2, m3, m4, mdc, mpr, batch_idx):
    out7, out14 = _roi_align_pallas(feat, boxes, batch_idx)
    flat = out7.transpose(0, 3, 1, 2).reshape(out7.shape[0], -1)
    box_result = jax.nn.relu(jax.nn.relu(flat @ bw1) @ bw2) @ bwo
    cls_result = jax.nn.relu(jax.nn.relu(flat @ cw1) @ cw2) @ cwo
    x = out14.astype(jnp.float32).transpose(0, 3, 1, 2)
    for w in (m1, m2, m3, m4):
        x = jax.nn.relu(_conv(x, w))
    x = jax.nn.relu(lax.conv_transpose(x, mdc, (2, 2), 'VALID',
                                       dimension_numbers=('NCHW', 'IOHW', 'NCHW')))
    mask_result = _conv(x, mpr)
    return (box_result, cls_result, mask_result)
